# baseline (speedup 1.0000x reference)
import functools

import jax
import jax.numpy as jnp
from jax import lax
from jax.experimental import pallas as pl
from jax.experimental.pallas import tpu as pltpu
from jax.experimental.pallas import tpu_sc as plsc

N_NODES = 10000
N_EDGES = 320000
D = 128
LANES = 16
NUM_CORES = 2
NUM_SUBCORES = 16
NUM_WORKERS = NUM_CORES * NUM_SUBCORES
EDGES_PER_WORKER = N_EDGES // NUM_WORKERS
CHUNK = 80
CHUNKS_PER_WORKER = EDGES_PER_WORKER // CHUNK
ROWS_PER_SUBCORE = 624
TAIL_ROWS = 16
ZROWS = 16


_GATHER_DNUMS = lax.GatherDimensionNumbers(
    offset_dims=(), collapsed_slice_dims=(0,), start_index_map=(0,))


def _lane_broadcast(vec, i):
    idx = jnp.full((LANES, 1), i, jnp.int32)
    return lax.gather(vec, idx, _GATHER_DNUMS, slice_sizes=(1,),
                      mode=lax.GatherScatterMode.PROMISE_IN_BOUNDS)


def _spmm_body(rc_hbm, vals_hbm, feat_hbm, out_hbm,
               rcv, colv, rowv, valv, gbuf, acc, sem, isem, ssem):
    cid = lax.axis_index("c")
    sid = lax.axis_index("s")
    wid = sid * NUM_CORES + cid

    def issue_idx(k):
        s = k % 4
        e0 = wid * EDGES_PER_WORKER + k * CHUNK
        pltpu.async_copy(rc_hbm.at[pl.ds(e0, CHUNK)], rcv.at[s], isem)
        pltpu.async_copy(vals_hbm.at[pl.ds(e0, CHUNK)], valv.at[s], isem)

    def wait_idx(k):
        s = k % 4
        e0 = wid * EDGES_PER_WORKER + k * CHUNK
        pltpu.make_async_copy(rc_hbm.at[pl.ds(e0, CHUNK)], rcv.at[s],
                              isem).wait()
        pltpu.make_async_copy(vals_hbm.at[pl.ds(e0, CHUNK)], valv.at[s],
                              isem).wait()

    def unpack_idx(k):
        s = k % 4
        low = jnp.int32(0xFFFF)
        sixteen = jnp.int32(16)
        for g in range(CHUNK // LANES):
            sl = pl.ds(g * LANES, LANES)
            w = rcv[s, sl]
            colv[s, sl] = jnp.bitwise_and(w, low)
            rowv[s, sl] = jnp.right_shift(w, sixteen)

    def issue_gather(k):
        pltpu.async_copy(feat_hbm.at[colv.at[k % 4]], gbuf.at[k % 3], sem)

    def wait_gather(k):
        pltpu.make_async_copy(feat_hbm.at[colv.at[k % 4]],
                              gbuf.at[k % 3], sem).wait()

    def issue_scatter(k):
        pltpu.async_copy(gbuf.at[k % 3], acc.at[rowv.at[k % 4]], ssem,
                         add=True)

    def wait_scatter(k):
        pltpu.make_async_copy(gbuf.at[k % 3], acc.at[rowv.at[k % 4]],
                              ssem).wait()

    issue_idx(0)
    issue_idx(1)
    issue_idx(2)

    zero = jnp.zeros((LANES,), jnp.float32)
    for r in range(ZROWS):
        for j in range(D // LANES):
            gbuf[0, r, pl.ds(j * LANES, LANES)] = zero

    def zero_issue(k, _):
        pltpu.async_copy(gbuf.at[0, pl.ds(0, ZROWS)],
                         acc.at[pl.ds(sid * ROWS_PER_SUBCORE + k * ZROWS, ZROWS)],
                         ssem)
        return 0

    def zero_wait(k, _):
        pltpu.make_async_copy(
            gbuf.at[0, pl.ds(0, ZROWS)],
            acc.at[pl.ds(sid * ROWS_PER_SUBCORE + k * ZROWS, ZROWS)],
            ssem).wait()
        return 0

    n_zero = ROWS_PER_SUBCORE // ZROWS + jnp.where(sid == NUM_SUBCORES - 1, 1, 0)
    lax.fori_loop(0, n_zero, zero_issue, 0)
    lax.fori_loop(0, n_zero, zero_wait, 0)
    plsc.subcore_barrier()

    wait_idx(0)
    unpack_idx(0)
    issue_gather(0)
    wait_idx(1)
    unpack_idx(1)
    issue_gather(1)

    def chunk_step(k, _):
        @pl.when(k + 3 < CHUNKS_PER_WORKER)
        def _issue_idx_k3():
            issue_idx(k + 3)

        slot = k % 3
        wait_gather(k)
        for g in range(CHUNK // LANES):
            vvec = valv[k % 4, pl.ds(g * LANES, LANES)]
            svs = [_lane_broadcast(vvec, i) for i in range(LANES)]
            for i in range(LANES):
                e = g * LANES + i
                sls = [pl.ds(j * LANES, LANES) for j in range(D // LANES)]
                loaded = [gbuf[slot, e, sl] for sl in sls]
                for sl, x in zip(sls, loaded):
                    gbuf[slot, e, sl] = x * svs[i]

        @pl.when(k >= 1)
        def _wait_scatter_km1():
            wait_scatter(k - 1)

        issue_scatter(k)

        @pl.when(k + 2 < CHUNKS_PER_WORKER)
        def _issue_gather_k2():
            wait_idx(k + 2)
            unpack_idx(k + 2)
            issue_gather(k + 2)

        return 0

    lax.fori_loop(0, CHUNKS_PER_WORKER, chunk_step, 0)
    wait_scatter(CHUNKS_PER_WORKER - 1)
    plsc.subcore_barrier()

    r0 = sid * ROWS_PER_SUBCORE
    pltpu.sync_copy(acc.at[pl.ds(r0, ROWS_PER_SUBCORE)],
                    out_hbm.at[cid, pl.ds(r0, ROWS_PER_SUBCORE)])

    @pl.when(sid == NUM_SUBCORES - 1)
    def _flush_tail():
        t0 = NUM_SUBCORES * ROWS_PER_SUBCORE
        pltpu.sync_copy(acc.at[pl.ds(t0, TAIL_ROWS)],
                        out_hbm.at[cid, pl.ds(t0, TAIL_ROWS)])


_spmm = functools.partial(
    pl.kernel,
    out_type=jax.ShapeDtypeStruct((NUM_CORES, N_NODES, D), jnp.float32),
    mesh=plsc.VectorSubcoreMesh(core_axis_name="c", subcore_axis_name="s"),
    scratch_types=[
        pltpu.VMEM((4, CHUNK), jnp.int32),
        pltpu.VMEM((4, CHUNK), jnp.int32),
        pltpu.VMEM((4, CHUNK), jnp.int32),
        pltpu.VMEM((4, CHUNK), jnp.float32),
        pltpu.VMEM((3, CHUNK, D), jnp.float32),
        pltpu.VMEM_SHARED((N_NODES, D), jnp.float32),
        pltpu.SemaphoreType.DMA,
        pltpu.SemaphoreType.DMA,
        pltpu.SemaphoreType.DMA,
    ],
)(_spmm_body)


ROW_BLOCK = 2000


def _finish_body(acc_ref, w_ref, b_ref, gamma_ref, beta_ref, o_ref):
    x = acc_ref[0] + acc_ref[1]
    y = jnp.dot(x, w_ref[...], preferred_element_type=jnp.float32,
                precision=lax.Precision.HIGHEST) + b_ref[...]
    h = jnp.where(y > 0, y, jnp.exp(y) - 1.0)
    mean = jnp.mean(h, axis=-1, keepdims=True)
    c = h - mean
    var = jnp.mean(c * c, axis=-1, keepdims=True)
    o_ref[...] = c * lax.rsqrt(var + 1e-5) * gamma_ref[...] + beta_ref[...]


def _finish(acc, w, b, gamma, beta):
    grid = (N_NODES // ROW_BLOCK,)
    return pl.pallas_call(
        _finish_body,
        grid=grid,
        in_specs=[
            pl.BlockSpec((NUM_CORES, ROW_BLOCK, D), lambda i: (0, i, 0)),
            pl.BlockSpec((D, D), lambda i: (0, 0)),
            pl.BlockSpec((1, D), lambda i: (0, 0)),
            pl.BlockSpec((1, D), lambda i: (0, 0)),
            pl.BlockSpec((1, D), lambda i: (0, 0)),
        ],
        out_specs=pl.BlockSpec((ROW_BLOCK, D), lambda i: (i, 0)),
        out_shape=jax.ShapeDtypeStruct((N_NODES, D), jnp.float32),
    )(acc, w, b, gamma, beta)


def kernel(adj_indices, adj_values, features, W, b, gamma, beta):
    rc = adj_indices[0] * 65536 + adj_indices[1]
    part = _spmm(rc, adj_values, features)
    return _finish(part, W, b, gamma.reshape(1, D), beta.reshape(1, D))


# speedup vs baseline: 1.0056x; 1.0056x over previous
import functools

import jax
import jax.numpy as jnp
from jax import lax
from jax.experimental import pallas as pl
from jax.experimental.pallas import tpu as pltpu
from jax.experimental.pallas import tpu_sc as plsc

N_NODES = 10000
N_EDGES = 320000
D = 128
LANES = 16
NUM_CORES = 2
NUM_SUBCORES = 16
NUM_WORKERS = NUM_CORES * NUM_SUBCORES
EDGES_PER_WORKER = N_EDGES // NUM_WORKERS
CHUNK = 80
CHUNKS_PER_WORKER = EDGES_PER_WORKER // CHUNK
ROWS_PER_SUBCORE = 624
TAIL_ROWS = 16
ZROWS = 16


_GATHER_DNUMS = lax.GatherDimensionNumbers(
    offset_dims=(), collapsed_slice_dims=(0,), start_index_map=(0,))


def _lane_broadcast(vec, i):
    idx = jnp.full((LANES, 1), i, jnp.int32)
    return lax.gather(vec, idx, _GATHER_DNUMS, slice_sizes=(1,),
                      mode=lax.GatherScatterMode.PROMISE_IN_BOUNDS)


def _spmm_body(rows_hbm, cols_hbm, vals_hbm, feat_hbm, out_hbm,
               colv, rowv, valv, gbuf, acc, sem, isem, ssem):
    cid = lax.axis_index("c")
    sid = lax.axis_index("s")
    wid = sid * NUM_CORES + cid

    def issue_idx(k):
        s = k % 4
        e0 = wid * EDGES_PER_WORKER + k * CHUNK
        pltpu.async_copy(cols_hbm.at[pl.ds(e0, CHUNK)], colv.at[s], isem)
        pltpu.async_copy(rows_hbm.at[pl.ds(e0, CHUNK)], rowv.at[s], isem)
        pltpu.async_copy(vals_hbm.at[pl.ds(e0, CHUNK)], valv.at[s], isem)

    def wait_idx(k):
        s = k % 4
        e0 = wid * EDGES_PER_WORKER + k * CHUNK
        pltpu.make_async_copy(cols_hbm.at[pl.ds(e0, CHUNK)], colv.at[s],
                              isem).wait()
        pltpu.make_async_copy(rows_hbm.at[pl.ds(e0, CHUNK)], rowv.at[s],
                              isem).wait()
        pltpu.make_async_copy(vals_hbm.at[pl.ds(e0, CHUNK)], valv.at[s],
                              isem).wait()

    def issue_gather(k):
        pltpu.async_copy(feat_hbm.at[colv.at[k % 4]], gbuf.at[k % 3], sem)

    def wait_gather(k):
        pltpu.make_async_copy(feat_hbm.at[colv.at[k % 4]],
                              gbuf.at[k % 3], sem).wait()

    def issue_scatter(k):
        pltpu.async_copy(gbuf.at[k % 3], acc.at[rowv.at[k % 4]], ssem,
                         add=True)

    def wait_scatter(k):
        pltpu.make_async_copy(gbuf.at[k % 3], acc.at[rowv.at[k % 4]],
                              ssem).wait()

    issue_idx(0)
    issue_idx(1)
    issue_idx(2)

    zero = jnp.zeros((LANES,), jnp.float32)
    for r in range(ZROWS):
        for j in range(D // LANES):
            gbuf[0, r, pl.ds(j * LANES, LANES)] = zero

    def zero_issue(k, _):
        pltpu.async_copy(gbuf.at[0, pl.ds(0, ZROWS)],
                         acc.at[pl.ds(sid * ROWS_PER_SUBCORE + k * ZROWS, ZROWS)],
                         ssem)
        return 0

    def zero_wait(k, _):
        pltpu.make_async_copy(
            gbuf.at[0, pl.ds(0, ZROWS)],
            acc.at[pl.ds(sid * ROWS_PER_SUBCORE + k * ZROWS, ZROWS)],
            ssem).wait()
        return 0

    n_zero = ROWS_PER_SUBCORE // ZROWS + jnp.where(sid == NUM_SUBCORES - 1, 1, 0)
    lax.fori_loop(0, n_zero, zero_issue, 0)
    lax.fori_loop(0, n_zero, zero_wait, 0)
    plsc.subcore_barrier()

    wait_idx(0)
    issue_gather(0)
    wait_idx(1)
    issue_gather(1)

    def chunk_step(k, _):
        @pl.when(k + 3 < CHUNKS_PER_WORKER)
        def _issue_idx_k3():
            issue_idx(k + 3)

        slot = k % 3
        wait_gather(k)
        for g in range(CHUNK // LANES):
            vvec = valv[k % 4, pl.ds(g * LANES, LANES)]
            svs = [_lane_broadcast(vvec, i) for i in range(LANES)]
            for i in range(LANES):
                e = g * LANES + i
                sls = [pl.ds(j * LANES, LANES) for j in range(D // LANES)]
                loaded = [gbuf[slot, e, sl] for sl in sls]
                for sl, x in zip(sls, loaded):
                    gbuf[slot, e, sl] = x * svs[i]

        @pl.when(k >= 1)
        def _wait_scatter_km1():
            wait_scatter(k - 1)

        issue_scatter(k)

        @pl.when(k + 2 < CHUNKS_PER_WORKER)
        def _issue_gather_k2():
            wait_idx(k + 2)
            issue_gather(k + 2)

        return 0

    lax.fori_loop(0, CHUNKS_PER_WORKER, chunk_step, 0)
    wait_scatter(CHUNKS_PER_WORKER - 1)
    plsc.subcore_barrier()

    r0 = sid * ROWS_PER_SUBCORE
    pltpu.sync_copy(acc.at[pl.ds(r0, ROWS_PER_SUBCORE)],
                    out_hbm.at[cid, pl.ds(r0, ROWS_PER_SUBCORE)])

    @pl.when(sid == NUM_SUBCORES - 1)
    def _flush_tail():
        t0 = NUM_SUBCORES * ROWS_PER_SUBCORE
        pltpu.sync_copy(acc.at[pl.ds(t0, TAIL_ROWS)],
                        out_hbm.at[cid, pl.ds(t0, TAIL_ROWS)])


_spmm = functools.partial(
    pl.kernel,
    out_type=jax.ShapeDtypeStruct((NUM_CORES, N_NODES, D), jnp.float32),
    mesh=plsc.VectorSubcoreMesh(core_axis_name="c", subcore_axis_name="s"),
    scratch_types=[
        pltpu.VMEM((4, CHUNK), jnp.int32),
        pltpu.VMEM((4, CHUNK), jnp.int32),
        pltpu.VMEM((4, CHUNK), jnp.float32),
        pltpu.VMEM((3, CHUNK, D), jnp.float32),
        pltpu.VMEM_SHARED((N_NODES, D), jnp.float32),
        pltpu.SemaphoreType.DMA,
        pltpu.SemaphoreType.DMA,
        pltpu.SemaphoreType.DMA,
    ],
)(_spmm_body)


ROW_BLOCK = 2000


def _finish_body(acc_ref, w_ref, b_ref, gamma_ref, beta_ref, o_ref):
    x = acc_ref[0] + acc_ref[1]
    y = jnp.dot(x, w_ref[...], preferred_element_type=jnp.float32,
                precision=lax.Precision.HIGHEST) + b_ref[...]
    h = jnp.where(y > 0, y, jnp.exp(y) - 1.0)
    mean = jnp.mean(h, axis=-1, keepdims=True)
    c = h - mean
    var = jnp.mean(c * c, axis=-1, keepdims=True)
    o_ref[...] = c * lax.rsqrt(var + 1e-5) * gamma_ref[...] + beta_ref[...]


def _finish(acc, w, b, gamma, beta):
    grid = (N_NODES // ROW_BLOCK,)
    return pl.pallas_call(
        _finish_body,
        grid=grid,
        in_specs=[
            pl.BlockSpec((NUM_CORES, ROW_BLOCK, D), lambda i: (0, i, 0)),
            pl.BlockSpec((D, D), lambda i: (0, 0)),
            pl.BlockSpec((1, D), lambda i: (0, 0)),
            pl.BlockSpec((1, D), lambda i: (0, 0)),
            pl.BlockSpec((1, D), lambda i: (0, 0)),
        ],
        out_specs=pl.BlockSpec((ROW_BLOCK, D), lambda i: (i, 0)),
        out_shape=jax.ShapeDtypeStruct((N_NODES, D), jnp.float32),
    )(acc, w, b, gamma, beta)


def kernel(adj_indices, adj_values, features, W, b, gamma, beta):
    rows = adj_indices[0]
    cols = adj_indices[1]
    part = _spmm(rows, cols, adj_values, features)
    return _finish(part, W, b, gamma.reshape(1, D), beta.reshape(1, D))
